# baseline (speedup 1.0000x reference)
import dataclasses

import jax
import jax.numpy as jnp
from jax import lax
from jax.experimental import pallas as pl
from jax.experimental.pallas import tpu as pltpu
from jax.experimental.pallas import tpu_sc as plsc

_N = 10000
_DEG = 32
_F = 128
_OUT_F = 128
_E = _N * _DEG

_NW = 32
_CH = 4
_EPC = _CH * _DEG
_NCHUNKS = _N // _CH
_CPW = 80
_NCH_PAD = _CPW * _NW
_RPW = _CPW * _CH
_TAIL = _NCHUNKS - (_NW - 1) * _CPW

_LANES = 16
_FCH = _F // _LANES


def _agg_body(idx_hbm, val_hbm, x_hbm, y_hbm,
              idx_v, val_v, rows0, rows1, out_v, sem0, sem1):
    wid = lax.axis_index("c") * 16 + lax.axis_index("s")
    c0 = wid * _CPW

    pltpu.sync_copy(idx_hbm.at[pl.ds(c0, _CPW)], idx_v)
    pltpu.sync_copy(val_hbm.at[pl.ds(c0, _CPW)], val_v)

    def start(slot, buf, sem):
        pltpu.async_copy(x_hbm.at[idx_v.at[slot]], buf, sem)

    def wait(buf, sem):
        pltpu.make_async_copy(x_hbm.at[idx_v.at[0]], buf, sem).wait()

    def compute(slot, buf):
        for n in range(_CH):
            def edge(e, accs, n=n):
                j = n * _DEG + e
                v = plsc.load_gather(
                    val_v,
                    [jnp.full((_LANES,), slot, jnp.int32),
                     jnp.full((_LANES,), j, jnp.int32)])
                return tuple(
                    accs[fc] + v * buf[j, pl.ds(fc * _LANES, _LANES)]
                    for fc in range(_FCH))

            accs = lax.fori_loop(
                0, _DEG, edge,
                tuple(jnp.zeros((_LANES,), jnp.float32)
                      for _ in range(_FCH)))
            for fc in range(_FCH):
                out_v[slot * _CH + n, pl.ds(fc * _LANES, _LANES)] = accs[fc]

    @pl.loop(0, _CPW)
    def _(g):
        start(g, rows0, sem0)
        wait(rows0, sem0)
        compute(g, rows0)

    tail_rows = _TAIL * _CH
    pltpu.sync_copy(out_v.at[pl.ds(0, tail_rows)],
                    y_hbm.at[pl.ds(c0 * _CH, tail_rows)])

    @pl.when(wid < _NW - 1)
    def _():
        pltpu.sync_copy(out_v.at[pl.ds(tail_rows, _RPW - tail_rows)],
                        y_hbm.at[pl.ds(c0 * _CH + tail_rows,
                                       _RPW - tail_rows)])


@jax.jit
def _aggregate(col_idx, values, X):
    pad = _NCH_PAD * _EPC - _E
    idx2d = jnp.pad(col_idx, (0, pad)).reshape(_NCH_PAD, _EPC)
    val2d = jnp.pad(values, (0, pad)).reshape(_NCH_PAD, _EPC)

    mesh = plsc.VectorSubcoreMesh(core_axis_name="c", subcore_axis_name="s")
    cp = pltpu.CompilerParams()
    if "needs_layout_passes" in pltpu.CompilerParams.__dataclass_fields__:
        cp = dataclasses.replace(cp, needs_layout_passes=False)
    return pl.kernel(
        _agg_body,
        out_type=jax.ShapeDtypeStruct((_N, _F), jnp.float32),
        mesh=mesh,
        scratch_types=[
            pltpu.VMEM((_CPW, _EPC), jnp.int32),
            pltpu.VMEM((_CPW, _EPC), jnp.float32),
            pltpu.VMEM((_EPC, _F), jnp.float32),
            pltpu.VMEM((_EPC, _F), jnp.float32),
            pltpu.VMEM((_RPW, _F), jnp.float32),
            pltpu.SemaphoreType.DMA,
            pltpu.SemaphoreType.DMA,
        ],
        compiler_params=cp,
    )(idx2d, val2d, X)


def _mm_body(y_ref, w_ref, o_ref):
    o_ref[...] = jnp.dot(y_ref[...], w_ref[...],
                         preferred_element_type=jnp.float32,
                         precision=lax.Precision.HIGHEST)


_MB = 2000


@jax.jit
def _matmul(Y, W):
    return pl.pallas_call(
        _mm_body,
        grid=(_N // _MB,),
        in_specs=[
            pl.BlockSpec((_MB, _F), lambda i: (i, 0)),
            pl.BlockSpec((_F, _OUT_F), lambda i: (0, 0)),
        ],
        out_specs=pl.BlockSpec((_MB, _OUT_F), lambda i: (i, 0)),
        out_shape=jax.ShapeDtypeStruct((_N, _OUT_F), jnp.float32),
    )(Y, W)


def kernel(row_ptr, col_idx, values, X, num_neighbors, W):
    Y = _aggregate(col_idx, values, X)
    return _matmul(Y, W)


# speedup vs baseline: 1.0341x; 1.0341x over previous
import dataclasses

import jax
import jax.numpy as jnp
from jax import lax
from jax.experimental import pallas as pl
from jax.experimental.pallas import tpu as pltpu
from jax.experimental.pallas import tpu_sc as plsc

_N = 10000
_DEG = 32
_F = 128
_OUT_F = 128
_E = _N * _DEG

_NW = 32
_CH = 4
_EPC = _CH * _DEG
_NITER = 80
_NCH_PAD = _NITER * _NW
_N_PAD = _NCH_PAD * _CH

_LANES = 16
_FCH = _F // _LANES


def _agg_body(idx_hbm, val_hbm, x_hbm, y_hbm,
              idx0, val0, rows0, out0, semi0, semv0, semg0, semo0,
              idx1, val1, rows1, out1, semi1, semv1, semg1, semo1):
    wid = lax.axis_index("s") * 2 + lax.axis_index("c")

    def chunk_of(g):
        return jnp.minimum(g, _NITER - 1) * _NW + wid

    def start_i(g, idx_v, semi):
        pltpu.async_copy(idx_hbm.at[pl.ds(chunk_of(g) * _EPC, _EPC)],
                         idx_v, semi)

    def wait_i(idx_v, semi):
        pltpu.make_async_copy(idx_hbm.at[pl.ds(0, _EPC)], idx_v, semi).wait()

    def start_v(g, val_v, semv):
        pltpu.async_copy(val_hbm.at[pl.ds(chunk_of(g) * _EPC, _EPC)],
                         val_v, semv)

    def wait_v(val_v, semv):
        pltpu.make_async_copy(val_hbm.at[pl.ds(0, _EPC)], val_v, semv).wait()

    def start_g(idx_v, rows_v, semg):
        pltpu.async_copy(x_hbm.at[idx_v], rows_v, semg)

    def wait_g(idx_v, rows_v, semg):
        pltpu.make_async_copy(x_hbm.at[idx_v], rows_v, semg).wait()

    def start_o(g, out_v, semo):
        c = chunk_of(g)
        pltpu.async_copy(out_v, y_hbm.at[pl.ds(c * _CH, _CH)], semo)

    def wait_o(out_v, semo):
        pltpu.make_async_copy(out_v, y_hbm.at[pl.ds(0, _CH)], semo).wait()

    def compute(rows_v, val_v, out_v):
        for n in range(_CH):
            def edge(e, accs, n=n):
                j = n * _DEG + e
                v = plsc.load_gather(
                    val_v, [jnp.full((_LANES,), j, jnp.int32)])
                return tuple(
                    accs[fc] + v * rows_v[j, pl.ds(fc * _LANES, _LANES)]
                    for fc in range(_FCH))

            accs = lax.fori_loop(
                0, _DEG, edge,
                tuple(jnp.zeros((_LANES,), jnp.float32)
                      for _ in range(_FCH)))
            for fc in range(_FCH):
                out_v[n, pl.ds(fc * _LANES, _LANES)] = accs[fc]

    sets = ((idx0, val0, rows0, out0, semi0, semv0, semg0, semo0),
            (idx1, val1, rows1, out1, semi1, semv1, semg1, semo1))

    start_i(0, idx0, semi0)
    start_i(1, idx1, semi1)
    start_v(0, val0, semv0)
    start_v(1, val1, semv1)
    wait_i(idx0, semi0)
    start_g(idx0, rows0, semg0)
    start_o(_NITER - 1, out0, semo0)
    start_o(_NITER - 1, out1, semo1)

    def step(g, a, b):
        idx_a, val_a, rows_a, out_a, semi_a, semv_a, semg_a, semo_a = a
        idx_b, val_b, rows_b, out_b, semi_b, semv_b, semg_b, semo_b = b
        wait_i(idx_b, semi_b)
        start_g(idx_b, rows_b, semg_b)
        wait_g(idx_a, rows_a, semg_a)
        start_i(g + 2, idx_a, semi_a)
        wait_v(val_a, semv_a)
        wait_o(out_a, semo_a)
        compute(rows_a, val_a, out_a)
        start_o(g, out_a, semo_a)
        start_v(g + 2, val_a, semv_a)

    @pl.loop(0, _NITER, step=2)
    def _(g):
        step(g, sets[0], sets[1])
        step(g + 1, sets[1], sets[0])

    wait_g(idx0, rows0, semg0)
    wait_i(idx1, semi1)
    wait_v(val0, semv0)
    wait_v(val1, semv1)
    wait_o(out0, semo0)
    wait_o(out1, semo1)


@jax.jit
def _aggregate(col_idx, values, X):
    pad = _NCH_PAD * _EPC - _E
    idx1d = jnp.pad(col_idx, (0, pad))
    val1d = jnp.pad(values, (0, pad))

    mesh = plsc.VectorSubcoreMesh(core_axis_name="c", subcore_axis_name="s")
    cp = pltpu.CompilerParams()
    if "needs_layout_passes" in pltpu.CompilerParams.__dataclass_fields__:
        cp = dataclasses.replace(cp, needs_layout_passes=False)
    buf_set = [
        pltpu.VMEM((_EPC,), jnp.int32),
        pltpu.VMEM((_EPC,), jnp.float32),
        pltpu.VMEM((_EPC, _F), jnp.float32),
        pltpu.VMEM((_CH, _F), jnp.float32),
        pltpu.SemaphoreType.DMA,
        pltpu.SemaphoreType.DMA,
        pltpu.SemaphoreType.DMA,
        pltpu.SemaphoreType.DMA,
    ]
    y = pl.kernel(
        _agg_body,
        out_type=jax.ShapeDtypeStruct((_N_PAD, _F), jnp.float32),
        mesh=mesh,
        scratch_types=buf_set + buf_set,
        compiler_params=cp,
    )(idx1d, val1d, X)
    return y[:_N]


def _mm_body(y_ref, w_ref, o_ref):
    o_ref[...] = jnp.dot(y_ref[...], w_ref[...],
                         preferred_element_type=jnp.float32,
                         precision=lax.Precision.HIGHEST)


_MB = 2000


@jax.jit
def _matmul(Y, W):
    return pl.pallas_call(
        _mm_body,
        grid=(_N // _MB,),
        in_specs=[
            pl.BlockSpec((_MB, _F), lambda i: (i, 0)),
            pl.BlockSpec((_F, _OUT_F), lambda i: (0, 0)),
        ],
        out_specs=pl.BlockSpec((_MB, _OUT_F), lambda i: (i, 0)),
        out_shape=jax.ShapeDtypeStruct((_N, _OUT_F), jnp.float32),
    )(Y, W)


def kernel(row_ptr, col_idx, values, X, num_neighbors, W):
    Y = _aggregate(col_idx, values, X)
    return _matmul(Y, W)


# speedup vs baseline: 4.5811x; 4.4298x over previous
import dataclasses

import jax
import jax.numpy as jnp
from jax import lax
from jax.experimental import pallas as pl
from jax.experimental.pallas import tpu as pltpu
from jax.experimental.pallas import tpu_sc as plsc

_N = 10000
_DEG = 32
_F = 128
_OUT_F = 128
_E = _N * _DEG

_NW = 32
_CH = 4
_EPC = _CH * _DEG
_NCHUNKS = _N // _CH
_NITER = 80

_LANES = 16
_FCH = _F // _LANES


def _agg_body(idx_hbm, val_hbm, x_hbm, y_hbm,
              idx0, val0, rows0, out0, semi0, semv0, semg0, semo0,
              idx1, val1, rows1, out1, semi1, semv1, semg1, semo1):
    wid = lax.axis_index("s") * 2 + lax.axis_index("c")

    def chunk_of(g):
        return jnp.minimum(jnp.minimum(g, _NITER - 1) * _NW + wid,
                           _NCHUNKS - 1)

    def start_i(g, idx_v, semi):
        pltpu.async_copy(idx_hbm.at[pl.ds(chunk_of(g) * _EPC, _EPC)],
                         idx_v, semi)

    def wait_i(idx_v, semi):
        pltpu.make_async_copy(idx_hbm.at[pl.ds(0, _EPC)], idx_v, semi).wait()

    def start_v(g, val_v, semv):
        pltpu.async_copy(val_hbm.at[pl.ds(chunk_of(g) * _EPC, _EPC)],
                         val_v, semv)

    def wait_v(val_v, semv):
        pltpu.make_async_copy(val_hbm.at[pl.ds(0, _EPC)], val_v, semv).wait()

    def start_g(idx_v, rows_v, semg):
        pltpu.async_copy(x_hbm.at[idx_v], rows_v, semg)

    def wait_g(idx_v, rows_v, semg):
        pltpu.make_async_copy(x_hbm.at[idx_v], rows_v, semg).wait()

    def start_o(g, out_v, semo):
        c = chunk_of(g)
        pltpu.async_copy(out_v, y_hbm.at[pl.ds(c * _CH, _CH)], semo)

    def wait_o(out_v, semo):
        pltpu.make_async_copy(out_v, y_hbm.at[pl.ds(0, _CH)], semo).wait()

    def compute(rows_v, val_v, out_v):
        for n in range(_CH):
            def edge(e, accs, n=n):
                j = n * _DEG + e
                v = plsc.load_gather(
                    val_v, [jnp.full((_LANES,), j, jnp.int32)])
                return tuple(
                    accs[fc] + v * rows_v[j, pl.ds(fc * _LANES, _LANES)]
                    for fc in range(_FCH))

            accs = lax.fori_loop(
                0, _DEG, edge,
                tuple(jnp.zeros((_LANES,), jnp.float32)
                      for _ in range(_FCH)))
            for fc in range(_FCH):
                out_v[n, pl.ds(fc * _LANES, _LANES)] = accs[fc]

    sets = ((idx0, val0, rows0, out0, semi0, semv0, semg0, semo0),
            (idx1, val1, rows1, out1, semi1, semv1, semg1, semo1))

    start_i(0, idx0, semi0)
    start_i(1, idx1, semi1)
    start_v(0, val0, semv0)
    start_v(1, val1, semv1)
    wait_i(idx0, semi0)
    start_g(idx0, rows0, semg0)

    def step(g, a, b):
        idx_a, val_a, rows_a, out_a, semi_a, semv_a, semg_a, semo_a = a
        idx_b, val_b, rows_b, out_b, semi_b, semv_b, semg_b, semo_b = b
        wait_i(idx_b, semi_b)
        start_g(idx_b, rows_b, semg_b)
        wait_g(idx_a, rows_a, semg_a)
        start_i(g + 2, idx_a, semi_a)
        wait_v(val_a, semv_a)

        @pl.when(g >= 2)
        def _():
            wait_o(out_a, semo_a)

        compute(rows_a, val_a, out_a)
        start_o(g, out_a, semo_a)
        start_v(g + 2, val_a, semv_a)

    @pl.loop(0, _NITER, step=2)
    def _(g):
        step(g, sets[0], sets[1])
        step(g + 1, sets[1], sets[0])

    wait_g(idx0, rows0, semg0)
    wait_i(idx1, semi1)
    wait_v(val0, semv0)
    wait_v(val1, semv1)
    wait_o(out0, semo0)
    wait_o(out1, semo1)


@jax.jit
def _aggregate(col_idx, values, X):
    mesh = plsc.VectorSubcoreMesh(core_axis_name="c", subcore_axis_name="s")
    cp = pltpu.CompilerParams()
    if "needs_layout_passes" in pltpu.CompilerParams.__dataclass_fields__:
        cp = dataclasses.replace(cp, needs_layout_passes=False)
    buf_set = [
        pltpu.VMEM((_EPC,), jnp.int32),
        pltpu.VMEM((_EPC,), jnp.float32),
        pltpu.VMEM((_EPC, _F), jnp.float32),
        pltpu.VMEM((_CH, _F), jnp.float32),
        pltpu.SemaphoreType.DMA,
        pltpu.SemaphoreType.DMA,
        pltpu.SemaphoreType.DMA,
        pltpu.SemaphoreType.DMA,
    ]
    return pl.kernel(
        _agg_body,
        out_type=jax.ShapeDtypeStruct((_N, _F), jnp.float32),
        mesh=mesh,
        scratch_types=buf_set + buf_set,
        compiler_params=cp,
    )(col_idx, values, X)


def _mm_body(y_ref, w_ref, o_ref):
    o_ref[...] = jnp.dot(y_ref[...], w_ref[...],
                         preferred_element_type=jnp.float32,
                         precision=lax.Precision.HIGHEST)


_MB = 2000


@jax.jit
def _matmul(Y, W):
    return pl.pallas_call(
        _mm_body,
        grid=(_N // _MB,),
        in_specs=[
            pl.BlockSpec((_MB, _F), lambda i: (i, 0)),
            pl.BlockSpec((_F, _OUT_F), lambda i: (0, 0)),
        ],
        out_specs=pl.BlockSpec((_MB, _OUT_F), lambda i: (i, 0)),
        out_shape=jax.ShapeDtypeStruct((_N, _OUT_F), jnp.float32),
    )(Y, W)


def kernel(row_ptr, col_idx, values, X, num_neighbors, W):
    Y = _aggregate(col_idx, values, X)
    return _matmul(Y, W)


# speedup vs baseline: 4.5938x; 1.0028x over previous
import dataclasses

import jax
import jax.numpy as jnp
from jax import lax
from jax.experimental import pallas as pl
from jax.experimental.pallas import tpu as pltpu
from jax.experimental.pallas import tpu_sc as plsc

_N = 10000
_DEG = 32
_F = 128
_OUT_F = 128
_E = _N * _DEG

_NW = 32
_CH = 4
_EPC = _CH * _DEG
_NCHUNKS = _N // _CH
_NITER = 80

_LANES = 16
_FCH = _F // _LANES


def _agg_body(idx_hbm, val_hbm, x_hbm, y_hbm,
              idx0, val0, rows0, out0, semi0, semv0, semg0, semo0,
              idx1, val1, rows1, out1, semi1, semv1, semg1, semo1):
    wid = lax.axis_index("s") * 2 + lax.axis_index("c")

    def chunk_of(g):
        return jnp.minimum(jnp.minimum(g, _NITER - 1) * _NW + wid,
                           _NCHUNKS - 1)

    def start_i(g, idx_v, semi):
        pltpu.async_copy(idx_hbm.at[pl.ds(chunk_of(g) * _EPC, _EPC)],
                         idx_v, semi)

    def wait_i(idx_v, semi):
        pltpu.make_async_copy(idx_hbm.at[pl.ds(0, _EPC)], idx_v, semi).wait()

    def start_v(g, val_v, semv):
        pltpu.async_copy(val_hbm.at[pl.ds(chunk_of(g) * _EPC, _EPC)],
                         val_v, semv)

    def wait_v(val_v, semv):
        pltpu.make_async_copy(val_hbm.at[pl.ds(0, _EPC)], val_v, semv).wait()

    def start_g(idx_v, rows_v, semg):
        pltpu.async_copy(x_hbm.at[idx_v], rows_v, semg)

    def wait_g(idx_v, rows_v, semg):
        pltpu.make_async_copy(x_hbm.at[idx_v], rows_v, semg).wait()

    def start_o(g, out_v, semo):
        c = chunk_of(g)
        pltpu.async_copy(out_v, y_hbm.at[pl.ds(c * _CH, _CH)], semo)

    def wait_o(out_v, semo):
        pltpu.make_async_copy(out_v, y_hbm.at[pl.ds(0, _CH)], semo).wait()

    def compute(rows_v, val_v, out_v):
        for n in range(_CH):
            def group(h, accs, n=n):
                vv = val_v[pl.ds(n * _DEG + (h // 2) * _LANES, _LANES)]
                sub = (h % 2) * 8
                base = n * _DEG + h * 8
                for k in range(8):
                    lane = jnp.full((_LANES, 1), sub + k, jnp.int32)
                    v = lax.gather(
                        vv, lane,
                        dimension_numbers=lax.GatherDimensionNumbers(
                            offset_dims=(), collapsed_slice_dims=(0,),
                            start_index_map=(0,)),
                        slice_sizes=(1,),
                        mode=lax.GatherScatterMode.PROMISE_IN_BOUNDS)
                    j = base + k
                    accs = tuple(
                        accs[fc] + v * rows_v[j, pl.ds(fc * _LANES, _LANES)]
                        for fc in range(_FCH))
                return accs

            accs = lax.fori_loop(
                0, _DEG // 8, group,
                tuple(jnp.zeros((_LANES,), jnp.float32)
                      for _ in range(_FCH)))
            for fc in range(_FCH):
                out_v[n, pl.ds(fc * _LANES, _LANES)] = accs[fc]

    sets = ((idx0, val0, rows0, out0, semi0, semv0, semg0, semo0),
            (idx1, val1, rows1, out1, semi1, semv1, semg1, semo1))

    start_i(0, idx0, semi0)
    start_i(1, idx1, semi1)
    start_v(0, val0, semv0)
    start_v(1, val1, semv1)
    wait_i(idx0, semi0)
    start_g(idx0, rows0, semg0)

    def step(g, a, b):
        idx_a, val_a, rows_a, out_a, semi_a, semv_a, semg_a, semo_a = a
        idx_b, val_b, rows_b, out_b, semi_b, semv_b, semg_b, semo_b = b
        wait_i(idx_b, semi_b)
        start_g(idx_b, rows_b, semg_b)
        wait_g(idx_a, rows_a, semg_a)
        start_i(g + 2, idx_a, semi_a)
        wait_v(val_a, semv_a)

        @pl.when(g >= 2)
        def _():
            wait_o(out_a, semo_a)

        compute(rows_a, val_a, out_a)
        start_o(g, out_a, semo_a)
        start_v(g + 2, val_a, semv_a)

    @pl.loop(0, _NITER, step=2)
    def _(g):
        step(g, sets[0], sets[1])
        step(g + 1, sets[1], sets[0])

    wait_g(idx0, rows0, semg0)
    wait_i(idx1, semi1)
    wait_v(val0, semv0)
    wait_v(val1, semv1)
    wait_o(out0, semo0)
    wait_o(out1, semo1)


@jax.jit
def _aggregate(col_idx, values, X):
    mesh = plsc.VectorSubcoreMesh(core_axis_name="c", subcore_axis_name="s")
    cp = pltpu.CompilerParams()
    if "needs_layout_passes" in pltpu.CompilerParams.__dataclass_fields__:
        cp = dataclasses.replace(cp, needs_layout_passes=False)
    buf_set = [
        pltpu.VMEM((_EPC,), jnp.int32),
        pltpu.VMEM((_EPC,), jnp.float32),
        pltpu.VMEM((_EPC, _F), jnp.float32),
        pltpu.VMEM((_CH, _F), jnp.float32),
        pltpu.SemaphoreType.DMA,
        pltpu.SemaphoreType.DMA,
        pltpu.SemaphoreType.DMA,
        pltpu.SemaphoreType.DMA,
    ]
    return pl.kernel(
        _agg_body,
        out_type=jax.ShapeDtypeStruct((_N, _F), jnp.float32),
        mesh=mesh,
        scratch_types=buf_set + buf_set,
        compiler_params=cp,
    )(col_idx, values, X)


def _mm_body(y_ref, w_ref, o_ref):
    o_ref[...] = jnp.dot(y_ref[...], w_ref[...],
                         preferred_element_type=jnp.float32,
                         precision=lax.Precision.HIGHEST)


_MB = 2000


@jax.jit
def _matmul(Y, W):
    return pl.pallas_call(
        _mm_body,
        grid=(_N // _MB,),
        in_specs=[
            pl.BlockSpec((_MB, _F), lambda i: (i, 0)),
            pl.BlockSpec((_F, _OUT_F), lambda i: (0, 0)),
        ],
        out_specs=pl.BlockSpec((_MB, _OUT_F), lambda i: (i, 0)),
        out_shape=jax.ShapeDtypeStruct((_N, _OUT_F), jnp.float32),
    )(Y, W)


def kernel(row_ptr, col_idx, values, X, num_neighbors, W):
    Y = _aggregate(col_idx, values, X)
    return _matmul(Y, W)


# speedup vs baseline: 4.6035x; 1.0021x over previous
import dataclasses

import numpy as _np

import jax
import jax.numpy as jnp
from jax import lax
from jax.experimental import pallas as pl
from jax.experimental.pallas import tpu as pltpu
from jax.experimental.pallas import tpu_sc as plsc

_N = 10000
_DEG = 32
_F = 128
_OUT_F = 128
_E = _N * _DEG

_NW = 32
_CH = 4
_EPC = _CH * _DEG
_NCHUNKS = _N // _CH
_NITER = 80

_LANES = 16
_FCH = _F // _LANES


def _agg_body(idx_hbm, val_hbm, x_hbm, y_hbm,
              idx0, val0, rows0, out0, semi0, semv0, semg0, semo0,
              idx1, val1, rows1, out1, semi1, semv1, semg1, semo1):
    wid = lax.axis_index("s") * 2 + lax.axis_index("c")

    def chunk_of(g):
        return jnp.minimum(jnp.minimum(g, _NITER - 1) * _NW + wid,
                           _NCHUNKS - 1)

    def start_i(g, idx_v, semi):
        pltpu.async_copy(idx_hbm.at[pl.ds(chunk_of(g) * _EPC, _EPC)],
                         idx_v, semi)

    def wait_i(idx_v, semi):
        pltpu.make_async_copy(idx_hbm.at[pl.ds(0, _EPC)], idx_v, semi).wait()

    def start_v(g, val_v, semv):
        pltpu.async_copy(val_hbm.at[pl.ds(chunk_of(g) * _EPC, _EPC)],
                         val_v, semv)

    def wait_v(val_v, semv):
        pltpu.make_async_copy(val_hbm.at[pl.ds(0, _EPC)], val_v, semv).wait()

    def start_g(idx_v, rows_v, semg):
        pltpu.async_copy(x_hbm.at[idx_v], rows_v, semg)

    def wait_g(idx_v, rows_v, semg):
        pltpu.make_async_copy(x_hbm.at[idx_v], rows_v, semg).wait()

    def start_o(g, out_v, semo):
        c = chunk_of(g)
        pltpu.async_copy(out_v, y_hbm.at[pl.ds(c * _CH, _CH)], semo)

    def wait_o(out_v, semo):
        pltpu.make_async_copy(out_v, y_hbm.at[pl.ds(0, _CH)], semo).wait()

    def bcast_gather(src, idx):
        return lax.gather(
            src, idx,
            dimension_numbers=lax.GatherDimensionNumbers(
                offset_dims=(), collapsed_slice_dims=(0,),
                start_index_map=(0,)),
            slice_sizes=(1,),
            mode=lax.GatherScatterMode.PROMISE_IN_BOUNDS)

    def compute(rows_v, val_v, out_v):
        for n in range(_CH):
            def group(h, accs, n=n):
                vv = val_v[pl.ds(n * _DEG + (h // 2) * _LANES, _LANES)]
                sub = (h % 2) * 8
                base = n * _DEG + h * 8
                for k in range(8):
                    v = bcast_gather(
                        vv, jnp.full((_LANES, 1), sub + k, jnp.int32))
                    j = base + k
                    accs = tuple(
                        accs[fc] + v * rows_v[j, pl.ds(fc * _LANES, _LANES)]
                        for fc in range(_FCH))
                return accs

            accs = lax.fori_loop(
                0, _DEG // 8, group,
                tuple(jnp.zeros((_LANES,), jnp.float32)
                      for _ in range(_FCH)))
            for fc in range(_FCH):
                out_v[n, pl.ds(fc * _LANES, _LANES)] = accs[fc]

    sets = ((idx0, val0, rows0, out0, semi0, semv0, semg0, semo0),
            (idx1, val1, rows1, out1, semi1, semv1, semg1, semo1))

    start_i(0, idx0, semi0)
    start_i(1, idx1, semi1)
    start_v(0, val0, semv0)
    start_v(1, val1, semv1)
    wait_i(idx0, semi0)
    start_g(idx0, rows0, semg0)

    def step(g, a, b):
        idx_a, val_a, rows_a, out_a, semi_a, semv_a, semg_a, semo_a = a
        idx_b, val_b, rows_b, out_b, semi_b, semv_b, semg_b, semo_b = b
        wait_i(idx_b, semi_b)
        start_g(idx_b, rows_b, semg_b)
        wait_g(idx_a, rows_a, semg_a)
        start_i(g + 2, idx_a, semi_a)
        wait_v(val_a, semv_a)

        @pl.when(g >= 2)
        def _():
            wait_o(out_a, semo_a)

        compute(rows_a, val_a, out_a)
        start_o(g, out_a, semo_a)
        start_v(g + 2, val_a, semv_a)

    @pl.loop(0, _NITER, step=2)
    def _(g):
        step(g, sets[0], sets[1])
        step(g + 1, sets[1], sets[0])

    wait_g(idx0, rows0, semg0)
    wait_i(idx1, semi1)
    wait_v(val0, semv0)
    wait_v(val1, semv1)
    wait_o(out0, semo0)
    wait_o(out1, semo1)


@jax.jit
def _aggregate(col_idx, values, X):
    mesh = plsc.VectorSubcoreMesh(core_axis_name="c", subcore_axis_name="s")
    cp = pltpu.CompilerParams()
    if "needs_layout_passes" in pltpu.CompilerParams.__dataclass_fields__:
        cp = dataclasses.replace(cp, needs_layout_passes=False)
    buf_set = [
        pltpu.VMEM((_EPC,), jnp.int32),
        pltpu.VMEM((_EPC,), jnp.float32),
        pltpu.VMEM((_EPC, _F), jnp.float32),
        pltpu.VMEM((_CH, _F), jnp.float32),
        pltpu.SemaphoreType.DMA,
        pltpu.SemaphoreType.DMA,
        pltpu.SemaphoreType.DMA,
        pltpu.SemaphoreType.DMA,
    ]
    return pl.kernel(
        _agg_body,
        out_type=jax.ShapeDtypeStruct((_N, _F), jnp.float32),
        mesh=mesh,
        scratch_types=buf_set + buf_set,
        compiler_params=cp,
    )(col_idx, values, X)


def _mm_body(y_ref, w_ref, o_ref):
    o_ref[...] = jnp.dot(y_ref[...], w_ref[...],
                         preferred_element_type=jnp.float32,
                         precision=lax.Precision.HIGHEST)


_MB = 2000


@jax.jit
def _matmul(Y, W):
    return pl.pallas_call(
        _mm_body,
        grid=(_N // _MB,),
        in_specs=[
            pl.BlockSpec((_MB, _F), lambda i: (i, 0)),
            pl.BlockSpec((_F, _OUT_F), lambda i: (0, 0)),
        ],
        out_specs=pl.BlockSpec((_MB, _OUT_F), lambda i: (i, 0)),
        out_shape=jax.ShapeDtypeStruct((_N, _OUT_F), jnp.float32),
    )(Y, W)


def kernel(row_ptr, col_idx, values, X, num_neighbors, W):
    Y = _aggregate(col_idx, values, X)
    return _matmul(Y, W)
